# Initial kernel scaffold; baseline (speedup 1.0000x reference)
import jax
import jax.numpy as jnp
from jax.experimental import pallas as pl


def kernel(img_embs, cap_embs, cap_lens, ln_g, ln_b, w1, b1, w2, b2, scale):
    raise NotImplementedError("write your pallas kernel here")


# speedup vs baseline: 1.3316x; 1.3316x over previous
import math
import functools

import jax
import jax.numpy as jnp
from jax.experimental import pallas as pl
from jax.experimental.pallas import tpu as pltpu

_EPS = 1e-12


def _l2norm_last(x):
    n = jnp.sqrt(jnp.sum(x * x, axis=-1, keepdims=True))
    return x / jnp.maximum(n, _EPS)


def _body(num_keep, img_ref, cap_norm_ref, cap_glo_ref, wm_ref,
          ln_g_ref, ln_b_ref, w1_ref, b1_ref, w2t_ref, b2_ref,
          sims_ref, mask_ref, score_s, z_s):
    img = img_ref[...]
    BI = img.shape[0]
    n_cap = cap_glo_ref.shape[0]
    L_s = img.shape[1] - 1

    cls_tok = img[:, 0:1, :]
    sp = img[:, 1:, :]

    spn = _l2norm_last(sp)
    glo = _l2norm_last(jnp.mean(sp, axis=1))
    self_attn = jnp.sum(glo[:, None, :] * spn, axis=-1)
    cap_attn = jax.lax.dot_general(
        cap_glo_ref[...], spn, (((1,), (2,)), ((), ())),
        precision=jax.lax.Precision.HIGHEST,
        preferred_element_type=jnp.float32)
    score = self_attn[None] + cap_attn
    score_s[...] = score

    bits = jax.lax.bitcast_convert_type(score + 0.0, jnp.int32)
    sign_bit = jnp.int32(-2147483648)
    skey = jnp.where(bits < 0,
                     jnp.bitwise_xor(jnp.bitwise_not(bits), sign_bit),
                     bits)

    def radix_step(t, p_u):
        bit = jax.lax.shift_left(jnp.int32(1), jnp.int32(31) - t)
        cand_u = p_u | bit
        cand_s = cand_u ^ sign_bit
        cnt = jnp.sum((skey >= cand_s).astype(jnp.int32), axis=2,
                      keepdims=True)
        return jnp.where(cnt >= num_keep, cand_u, p_u)

    p_u = jax.lax.fori_loop(0, 32, radix_step,
                            jnp.zeros((n_cap, BI, 1), jnp.int32))
    thr_s = p_u ^ sign_bit

    gt = skey > thr_s
    eq = skey == thr_s
    n_gt = jnp.sum(gt.astype(jnp.float32), axis=2, keepdims=True)
    need = jnp.float32(num_keep) - n_gt
    eqf = eq.astype(jnp.float32)
    row_i = jax.lax.broadcasted_iota(jnp.int32, (L_s, L_s), 0)
    col_i = jax.lax.broadcasted_iota(jnp.int32, (L_s, L_s), 1)
    tri = (row_i < col_i).astype(jnp.float32)
    rank = jax.lax.dot_general(
        eqf, tri, (((2,), (0,)), ((), ())),
        preferred_element_type=jnp.float32)
    keep = jnp.logical_or(gt, jnp.logical_and(eq, rank < need))
    mask_ref[...] = keep.astype(jnp.float32)

    mu = jnp.mean(sp, axis=-1, keepdims=True)
    var = jnp.mean((sp - mu) ** 2, axis=-1, keepdims=True)
    xn = (sp - mu) / jnp.sqrt(var + 1e-5)
    xn = xn * ln_g_ref[...][None] + ln_b_ref[...][None]
    h = jax.lax.dot_general(
        xn, w1_ref[...], (((2,), (0,)), ((), ())),
        preferred_element_type=jnp.float32)
    h = h + b1_ref[...][None]
    h = 0.5 * h * (1.0 + jax.lax.erf(h * jnp.float32(1.0 / math.sqrt(2.0))))
    z = jax.lax.dot_general(
        w2t_ref[...], h, (((1,), (2,)), ((), ())),
        preferred_element_type=jnp.float32)
    z_s[...] = z + b2_ref[...][0][:, None, None]

    cap_iota = jax.lax.broadcasted_iota(jnp.int32, (BI, n_cap), 1)

    def cap_step(c, sims):
        m = mask_ref[c]
        sc = score_s[c]
        zc = z_s[...]
        nk = 1.0 - m
        mx = jnp.max(jnp.where(nk > 0, sc, -1e30), axis=1, keepdims=True)
        e = jnp.exp(sc - mx) * nk
        ew = e / jnp.sum(e, axis=1, keepdims=True)
        zm = jnp.where(m[None] > 0, zc, -1e30)
        mz = jnp.max(zm, axis=2, keepdims=True)
        e2 = jnp.exp(zm - mz) * m[None]
        w_attn = e2 / jnp.sum(e2, axis=2, keepdims=True)
        w_all = jnp.concatenate([w_attn, ew[None]], axis=0)
        pooled = jax.lax.dot_general(
            w_all, sp, (((2,), (1,)), ((1,), (0,))),
            preferred_element_type=jnp.float32)
        sel = jnp.concatenate([cls_tok, pooled], axis=1)
        seln = _l2norm_last(sel)
        simc = jax.lax.dot_general(
            cap_norm_ref[c], seln, (((1,), (2,)), ((), ())),
            preferred_element_type=jnp.float32)
        smax = jnp.max(simc, axis=2)
        col = jnp.sum(smax * wm_ref[c][:, None], axis=0)
        return sims + col[:, None] * (cap_iota == c).astype(jnp.float32)

    sims_ref[...] = jax.lax.fori_loop(
        0, n_cap, cap_step, jnp.zeros((BI, n_cap), jnp.float32))


def kernel(img_embs, cap_embs, cap_lens, ln_g, ln_b, w1, b1, w2, b2, scale):
    B_v, L_v, C = img_embs.shape
    n_cap, L_t, _ = cap_embs.shape
    hidden = w1.shape[1]
    keeped = w2.shape[1]
    L_s = L_v - 1
    num_keep = math.ceil(L_s * 0.5)

    cap_norm = _l2norm_last(cap_embs)
    word_ids = jnp.arange(L_t)
    w_mask = (word_ids[None, :] < cap_lens[:, None]).astype(cap_embs.dtype)
    n_word_f = cap_lens.astype(cap_embs.dtype)
    cap_mean = jnp.sum(cap_embs * w_mask[:, :, None], axis=1) / n_word_f[:, None]
    cap_glo = _l2norm_last(cap_mean)
    wm_scaled = w_mask / n_word_f[:, None]

    s = jnp.reshape(scale, ())
    w2t = (w2 * s).T
    b2s = jnp.reshape(b2 * s, (1, keeped))
    ln_g2 = jnp.reshape(ln_g, (1, C))
    ln_b2 = jnp.reshape(ln_b, (1, C))
    b1r = jnp.reshape(b1, (1, hidden))

    BI = 8
    grid = (B_v // BI,)
    full = lambda *shape: pl.BlockSpec(shape, lambda i: (0,) * len(shape))

    sims, mask = pl.pallas_call(
        functools.partial(_body, num_keep),
        grid=grid,
        in_specs=[
            pl.BlockSpec((BI, L_v, C), lambda i: (i, 0, 0)),
            full(n_cap, L_t, C),
            full(n_cap, C),
            full(n_cap, L_t),
            full(1, C),
            full(1, C),
            full(C, hidden),
            full(1, hidden),
            full(keeped, hidden),
            full(1, keeped),
        ],
        out_specs=[
            pl.BlockSpec((BI, n_cap), lambda i: (i, 0)),
            pl.BlockSpec((n_cap, BI, L_s), lambda i: (0, i, 0)),
        ],
        out_shape=[
            jax.ShapeDtypeStruct((B_v, n_cap), jnp.float32),
            jax.ShapeDtypeStruct((n_cap, B_v, L_s), jnp.float32),
        ],
        scratch_shapes=[
            pltpu.VMEM((n_cap, BI, L_s), jnp.float32),
            pltpu.VMEM((keeped, BI, L_s), jnp.float32),
        ],
        compiler_params=pltpu.CompilerParams(
            dimension_semantics=("parallel",)),
    )(img_embs, cap_norm, cap_glo, wm_scaled, ln_g2, ln_b2, w1, b1r,
      w2t, b2s)
    return sims, mask


# speedup vs baseline: 1.3988x; 1.0504x over previous
import math
import functools

import jax
import jax.numpy as jnp
from jax.experimental import pallas as pl
from jax.experimental.pallas import tpu as pltpu

_EPS = 1e-12


def _l2norm_last(x):
    n = jnp.sqrt(jnp.sum(x * x, axis=-1, keepdims=True))
    return x / jnp.maximum(n, _EPS)


def _body(num_keep, img_ref, cap_norm_ref, cap_glo_ref, wm_ref,
          ln_g_ref, ln_b_ref, w1_ref, b1_ref, w2t_ref, b2_ref,
          sims_ref, mask_ref, score_s, z_s):
    img = img_ref[...]
    BI = img.shape[0]
    n_cap = cap_glo_ref.shape[0]
    L_s = img.shape[1] - 1

    cls_tok = img[:, 0:1, :]
    sp = img[:, 1:, :]

    spn = _l2norm_last(sp)
    glo = _l2norm_last(jnp.mean(sp, axis=1))
    self_attn = jnp.sum(glo[:, None, :] * spn, axis=-1)
    cap_attn = jax.lax.dot_general(
        cap_glo_ref[...], spn, (((1,), (2,)), ((), ())),
        precision=jax.lax.Precision.HIGHEST,
        preferred_element_type=jnp.float32)
    score = self_attn[None] + cap_attn
    score_s[...] = score

    bits = jax.lax.bitcast_convert_type(score + 0.0, jnp.int32)
    sign_bit = jnp.int32(-2147483648)
    skey = jnp.where(bits < 0,
                     jnp.bitwise_xor(jnp.bitwise_not(bits), sign_bit),
                     bits)

    def radix_step(t, p_u):
        bit = jax.lax.shift_left(jnp.int32(1), jnp.int32(31) - t)
        cand_u = p_u | bit
        cand_s = cand_u ^ sign_bit
        cnt = jnp.sum((skey >= cand_s).astype(jnp.int32), axis=2,
                      keepdims=True)
        return jnp.where(cnt >= num_keep, cand_u, p_u)

    p_u = jax.lax.fori_loop(0, 32, radix_step,
                            jnp.zeros((n_cap, BI, 1), jnp.int32))
    thr_s = p_u ^ sign_bit

    gt = skey > thr_s
    eq = skey == thr_s
    n_gt = jnp.sum(gt.astype(jnp.float32), axis=2, keepdims=True)
    need = jnp.float32(num_keep) - n_gt
    eqf = eq.astype(jnp.float32)
    row_i = jax.lax.broadcasted_iota(jnp.int32, (L_s, L_s), 0)
    col_i = jax.lax.broadcasted_iota(jnp.int32, (L_s, L_s), 1)
    tri = (row_i < col_i).astype(jnp.float32)
    rank = jax.lax.dot_general(
        eqf, tri, (((2,), (0,)), ((), ())),
        preferred_element_type=jnp.float32)
    keep = jnp.logical_or(gt, jnp.logical_and(eq, rank < need))
    mask_ref[...] = keep.astype(jnp.float32)

    mu = jnp.mean(sp, axis=-1, keepdims=True)
    var = jnp.mean((sp - mu) ** 2, axis=-1, keepdims=True)
    xn = (sp - mu) / jnp.sqrt(var + 1e-5)
    xn = xn * ln_g_ref[...][None] + ln_b_ref[...][None]
    h = jax.lax.dot_general(
        xn, w1_ref[...], (((2,), (0,)), ((), ())),
        preferred_element_type=jnp.float32)
    h = h + b1_ref[...][None]
    h = 0.5 * h * (1.0 + jax.lax.erf(h * jnp.float32(1.0 / math.sqrt(2.0))))
    z = jax.lax.dot_general(
        w2t_ref[...], h, (((1,), (2,)), ((), ())),
        preferred_element_type=jnp.float32)
    z_s[...] = z + b2_ref[...][0][:, None, None]

    cap_iota = jax.lax.broadcasted_iota(jnp.int32, (BI, n_cap), 1)

    def cap_step(c, sims):
        m = mask_ref[c]
        sc = score_s[c]
        zc = z_s[...]
        nk = 1.0 - m
        mx = jnp.max(jnp.where(nk > 0, sc, -1e30), axis=1, keepdims=True)
        e = jnp.exp(sc - mx) * nk
        ew = e / jnp.sum(e, axis=1, keepdims=True)
        zm = jnp.where(m[None] > 0, zc, -1e30)
        mz = jnp.max(zm, axis=2, keepdims=True)
        e2 = jnp.exp(zm - mz) * m[None]
        w_attn = e2 / jnp.sum(e2, axis=2, keepdims=True)
        w_all = jnp.concatenate([w_attn, ew[None]], axis=0)
        pooled = jax.lax.dot_general(
            w_all, sp, (((2,), (1,)), ((1,), (0,))),
            preferred_element_type=jnp.float32)
        sel = jnp.concatenate([cls_tok, pooled], axis=1)
        seln = _l2norm_last(sel)
        simc = jax.lax.dot_general(
            cap_norm_ref[c], seln, (((1,), (2,)), ((), ())),
            preferred_element_type=jnp.float32)
        smax = jnp.max(simc, axis=2)
        col = jnp.sum(smax * wm_ref[c][:, None], axis=0)
        return sims + col[:, None] * (cap_iota == c).astype(jnp.float32)

    sims_ref[...] = jax.lax.fori_loop(
        0, n_cap, cap_step, jnp.zeros((BI, n_cap), jnp.float32))


def kernel(img_embs, cap_embs, cap_lens, ln_g, ln_b, w1, b1, w2, b2, scale):
    B_v, L_v, C = img_embs.shape
    n_cap, L_t, _ = cap_embs.shape
    hidden = w1.shape[1]
    keeped = w2.shape[1]
    L_s = L_v - 1
    num_keep = math.ceil(L_s * 0.5)

    cap_norm = _l2norm_last(cap_embs)
    word_ids = jnp.arange(L_t)
    w_mask = (word_ids[None, :] < cap_lens[:, None]).astype(cap_embs.dtype)
    n_word_f = cap_lens.astype(cap_embs.dtype)
    cap_mean = jnp.sum(cap_embs * w_mask[:, :, None], axis=1) / n_word_f[:, None]
    cap_glo = _l2norm_last(cap_mean)
    wm_scaled = w_mask / n_word_f[:, None]

    s = jnp.reshape(scale, ())
    w2t = (w2 * s).T
    b2s = jnp.reshape(b2 * s, (1, keeped))
    ln_g2 = jnp.reshape(ln_g, (1, C))
    ln_b2 = jnp.reshape(ln_b, (1, C))
    b1r = jnp.reshape(b1, (1, hidden))

    BI = 16
    grid = (B_v // BI,)
    full = lambda *shape: pl.BlockSpec(shape, lambda i: (0,) * len(shape))

    sims, mask = pl.pallas_call(
        functools.partial(_body, num_keep),
        grid=grid,
        in_specs=[
            pl.BlockSpec((BI, L_v, C), lambda i: (i, 0, 0)),
            full(n_cap, L_t, C),
            full(n_cap, C),
            full(n_cap, L_t),
            full(1, C),
            full(1, C),
            full(C, hidden),
            full(1, hidden),
            full(keeped, hidden),
            full(1, keeped),
        ],
        out_specs=[
            pl.BlockSpec((BI, n_cap), lambda i: (i, 0)),
            pl.BlockSpec((n_cap, BI, L_s), lambda i: (0, i, 0)),
        ],
        out_shape=[
            jax.ShapeDtypeStruct((B_v, n_cap), jnp.float32),
            jax.ShapeDtypeStruct((n_cap, B_v, L_s), jnp.float32),
        ],
        scratch_shapes=[
            pltpu.VMEM((n_cap, BI, L_s), jnp.float32),
            pltpu.VMEM((keeped, BI, L_s), jnp.float32),
        ],
        compiler_params=pltpu.CompilerParams(
            dimension_semantics=("parallel",)),
    )(img_embs, cap_norm, cap_glo, wm_scaled, ln_g2, ln_b2, w1, b1r,
      w2t, b2s)
    return sims, mask


# speedup vs baseline: 2.2055x; 1.5767x over previous
import math
import functools

import jax
import jax.numpy as jnp
from jax.experimental import pallas as pl
from jax.experimental.pallas import tpu as pltpu

_EPS = 1e-12


def _sum_last(x, precision=None):
    ones = jnp.ones((x.shape[-1], 1), jnp.float32)
    r = jax.lax.dot_general(x, ones, (((x.ndim - 1,), (0,)), ((), ())),
                            precision=precision,
                            preferred_element_type=jnp.float32)
    return r[..., 0]


def _l2norm_last(x):
    ssq = _sum_last(x * x, precision=jax.lax.Precision.HIGHEST)
    n = jnp.sqrt(ssq)[..., None]
    return x / jnp.maximum(n, _EPS)


def _body(num_keep, img_ref, cap_norm_ref, cap_glo_ref, wm_ref,
          ln_g_ref, ln_b_ref, w1_ref, b1_ref, w2t_ref, b2_ref,
          sims_ref, mask_ref, score_s, z_s):
    img = img_ref[...]
    BI = img.shape[0]
    n_cap = cap_glo_ref.shape[0]
    L_s = img.shape[1] - 1

    cls_tok = img[:, 0:1, :]
    sp = img[:, 1:, :]

    spn = _l2norm_last(sp)
    glo = _l2norm_last(jnp.mean(sp, axis=1))
    self_attn = jax.lax.dot_general(
        glo, spn, (((1,), (2,)), ((0,), (0,))),
        precision=jax.lax.Precision.HIGHEST,
        preferred_element_type=jnp.float32)
    cap_attn = jax.lax.dot_general(
        cap_glo_ref[...], spn, (((1,), (2,)), ((), ())),
        precision=jax.lax.Precision.HIGHEST,
        preferred_element_type=jnp.float32)
    score = self_attn[None] + cap_attn
    score_s[...] = score - jnp.max(score, axis=2, keepdims=True)

    bits = jax.lax.bitcast_convert_type(score + 0.0, jnp.int32)
    sign_bit = jnp.int32(-2147483648)
    skey = jnp.where(bits < 0,
                     jnp.bitwise_xor(jnp.bitwise_not(bits), sign_bit),
                     bits)

    def radix_step(t, p_u):
        bit = jax.lax.shift_left(jnp.int32(1), jnp.int32(31) - t)
        cand_u = p_u | bit
        cand_s = cand_u ^ sign_bit
        cnt = _sum_last((skey >= cand_s).astype(jnp.float32))[..., None]
        return jnp.where(cnt >= num_keep, cand_u, p_u)

    p_u = jax.lax.fori_loop(0, 32, radix_step,
                            jnp.zeros((n_cap, BI, 1), jnp.int32))
    thr_s = p_u ^ sign_bit

    gt = skey > thr_s
    eq = skey == thr_s
    n_gt = _sum_last(gt.astype(jnp.float32))[..., None]
    need = jnp.float32(num_keep) - n_gt
    eqf = eq.astype(jnp.float32)
    row_i = jax.lax.broadcasted_iota(jnp.int32, (L_s, L_s), 0)
    col_i = jax.lax.broadcasted_iota(jnp.int32, (L_s, L_s), 1)
    tri = (row_i < col_i).astype(jnp.float32)
    rank = jax.lax.dot_general(
        eqf, tri, (((2,), (0,)), ((), ())),
        preferred_element_type=jnp.float32)
    keep = jnp.logical_or(gt, jnp.logical_and(eq, rank < need))
    mask_ref[...] = keep.astype(jnp.float32)

    C = sp.shape[-1]
    inv_c = jnp.float32(1.0 / C)
    mu = (_sum_last(sp, precision=jax.lax.Precision.HIGHEST)
          * inv_c)[..., None]
    ex2 = (_sum_last(sp * sp, precision=jax.lax.Precision.HIGHEST)
           * inv_c)[..., None]
    var = ex2 - mu * mu
    xn = (sp - mu) / jnp.sqrt(var + 1e-5)
    xn = xn * ln_g_ref[...][None] + ln_b_ref[...][None]
    h = jax.lax.dot_general(
        xn, w1_ref[...], (((2,), (0,)), ((), ())),
        preferred_element_type=jnp.float32)
    h = h + b1_ref[...][None]
    h = 0.5 * h * (1.0 + jax.lax.erf(h * jnp.float32(1.0 / math.sqrt(2.0))))
    z = jax.lax.dot_general(
        w2t_ref[...], h, (((1,), (2,)), ((), ())),
        preferred_element_type=jnp.float32)
    z = z + b2_ref[...][0][:, None, None]
    z_s[...] = z - jnp.max(z, axis=2, keepdims=True)

    cap_iota = jax.lax.broadcasted_iota(jnp.int32, (BI, n_cap), 1)

    def cap_step(c, sims):
        m = mask_ref[c]
        sc = score_s[c]
        zc = z_s[...]
        nk = 1.0 - m
        e = jnp.exp(sc) * nk
        ew = e / _sum_last(e)[..., None]
        e2 = jnp.exp(zc) * m[None]
        w_attn = e2 / _sum_last(e2)[..., None]
        w_all = jnp.concatenate([w_attn, ew[None]], axis=0)
        pooled = jax.lax.dot_general(
            w_all, sp, (((2,), (1,)), ((1,), (0,))),
            preferred_element_type=jnp.float32)
        sel = jnp.concatenate([cls_tok, pooled], axis=1)
        seln = _l2norm_last(sel)
        simc = jax.lax.dot_general(
            cap_norm_ref[c], seln, (((1,), (2,)), ((), ())),
            preferred_element_type=jnp.float32)
        smax = jnp.max(simc, axis=2)
        col = jnp.sum(smax * wm_ref[c][:, None], axis=0)
        return sims + col[:, None] * (cap_iota == c).astype(jnp.float32)

    sims_ref[...] = jax.lax.fori_loop(
        0, n_cap, cap_step, jnp.zeros((BI, n_cap), jnp.float32))


def kernel(img_embs, cap_embs, cap_lens, ln_g, ln_b, w1, b1, w2, b2, scale):
    B_v, L_v, C = img_embs.shape
    n_cap, L_t, _ = cap_embs.shape
    hidden = w1.shape[1]
    keeped = w2.shape[1]
    L_s = L_v - 1
    num_keep = math.ceil(L_s * 0.5)

    cap_norm = _l2norm_last(cap_embs)
    word_ids = jnp.arange(L_t)
    w_mask = (word_ids[None, :] < cap_lens[:, None]).astype(cap_embs.dtype)
    n_word_f = cap_lens.astype(cap_embs.dtype)
    cap_mean = jnp.sum(cap_embs * w_mask[:, :, None], axis=1) / n_word_f[:, None]
    cap_glo = _l2norm_last(cap_mean)
    wm_scaled = w_mask / n_word_f[:, None]

    s = jnp.reshape(scale, ())
    w2t = (w2 * s).T
    b2s = jnp.reshape(b2 * s, (1, keeped))
    ln_g2 = jnp.reshape(ln_g, (1, C))
    ln_b2 = jnp.reshape(ln_b, (1, C))
    b1r = jnp.reshape(b1, (1, hidden))

    BI = 16
    grid = (B_v // BI,)
    full = lambda *shape: pl.BlockSpec(shape, lambda i: (0,) * len(shape))

    sims, mask = pl.pallas_call(
        functools.partial(_body, num_keep),
        grid=grid,
        in_specs=[
            pl.BlockSpec((BI, L_v, C), lambda i: (i, 0, 0)),
            full(n_cap, L_t, C),
            full(n_cap, C),
            full(n_cap, L_t),
            full(1, C),
            full(1, C),
            full(C, hidden),
            full(1, hidden),
            full(keeped, hidden),
            full(1, keeped),
        ],
        out_specs=[
            pl.BlockSpec((BI, n_cap), lambda i: (i, 0)),
            pl.BlockSpec((n_cap, BI, L_s), lambda i: (0, i, 0)),
        ],
        out_shape=[
            jax.ShapeDtypeStruct((B_v, n_cap), jnp.float32),
            jax.ShapeDtypeStruct((n_cap, B_v, L_s), jnp.float32),
        ],
        scratch_shapes=[
            pltpu.VMEM((n_cap, BI, L_s), jnp.float32),
            pltpu.VMEM((keeped, BI, L_s), jnp.float32),
        ],
        compiler_params=pltpu.CompilerParams(
            dimension_semantics=("parallel",)),
    )(img_embs, cap_norm, cap_glo, wm_scaled, ln_g2, ln_b2, w1, b1r,
      w2t, b2s)
    return sims, mask
